# baseline (speedup 1.0000x reference)
import functools

import jax
import jax.numpy as jnp
from jax import lax
from jax.experimental import pallas as pl
from jax.experimental.pallas import tpu as pltpu
from jax.experimental.pallas import tpu_sc as plsc

B = 1024
S = 200
SO = S + 1
V = 100000
D = 128
CD = 128
MAXLEN = 256
EPS = 1e-12

NW = 32
NCK = 8
HALF_ROWS = B * S // NCK
ROWS_PER_W = HALF_ROWS // NW
CHUNK = 400
NCHUNK = ROWS_PER_W // CHUNK
BB = 64
HB = B // NCK


def _sc_gather_half(ids_flat, table, half):
    mesh = plsc.VectorSubcoreMesh(core_axis_name="c", subcore_axis_name="s")

    @functools.partial(
        pl.kernel,
        mesh=mesh,
        out_type=jax.ShapeDtypeStruct((HALF_ROWS, D), jnp.float32),
        scratch_types=[
            pltpu.VMEM((ROWS_PER_W,), jnp.int32),
            pltpu.VMEM((CHUNK, D), jnp.float32),
            pltpu.VMEM((CHUNK, D), jnp.float32),
            pltpu.SemaphoreType.DMA,
            pltpu.SemaphoreType.DMA,
            pltpu.SemaphoreType.DMA,
            pltpu.SemaphoreType.DMA,
        ],
    )
    def k(ids_hbm, table_hbm, out_hbm, idx_v, rows0, rows1, g0, g1, s0, s1):
        cid = lax.axis_index("c")
        sid = lax.axis_index("s")
        wid = sid * 2 + cid
        base = wid * ROWS_PER_W
        bufs = (rows0, rows1)
        gsems = (g0, g1)
        ssems = (s0, s1)
        pltpu.sync_copy(
            ids_hbm.at[pl.ds(half * HALF_ROWS + base, ROWS_PER_W)], idx_v)
        cpg = [None, None]
        cps = [None, None]
        cpg[0] = pltpu.async_copy(
            table_hbm.at[idx_v.at[pl.ds(0, CHUNK)]], bufs[0], gsems[0])
        for c in range(NCHUNK):
            p = c % 2
            if c + 1 < NCHUNK:
                q = 1 - p
                if cps[q] is not None:
                    cps[q].wait()
                cpg[q] = pltpu.async_copy(
                    table_hbm.at[idx_v.at[pl.ds((c + 1) * CHUNK, CHUNK)]],
                    bufs[q], gsems[q])
            cpg[p].wait()
            cps[p] = pltpu.async_copy(
                bufs[p], out_hbm.at[pl.ds(base + c * CHUNK, CHUNK)], ssems[p])
        cps[0].wait()
        cps[1].wait()

    return k(ids_flat, table)


def _tc_body(*refs):
    if len(refs) == 9:
        refs = refs[1:]
    g_ref, cond_ref, pos_ref, wc_ref, bc_ref, sc_ref, bi_ref, o_ref = refs
    g = g_ref[...]
    cond = cond_ref[:, 0, :]
    ce = jnp.dot(cond, wc_ref[...],
                 preferred_element_type=jnp.float32) + bc_ref[...]
    x = jnp.concatenate(
        [jnp.zeros((BB, 1, D), jnp.float32), g], axis=1)
    x = x + pos_ref[0, :SO, :][None, :, :] + ce[:, None, :]
    mean = jnp.mean(x, axis=-1, keepdims=True)
    var = jnp.mean(jnp.square(x), axis=-1, keepdims=True) - jnp.square(mean)
    y = (x - mean) * lax.rsqrt(var + EPS)
    o_ref[...] = y * sc_ref[...][None, None, :] + bi_ref[...][None, None, :]


def _tc_epilogue_half(o_prev, gathered, condition, pos_emb, W_c, b_c,
                      ln_scale, ln_bias, half):
    hb0 = half * (HB // BB)
    specs = [
        pl.BlockSpec((BB, S, D), lambda i: (i, 0, 0)),
        pl.BlockSpec((BB, 1, CD), lambda i: (hb0 + i, 0, 0)),
        pl.BlockSpec((1, MAXLEN, D), lambda i: (0, 0, 0)),
        pl.BlockSpec((CD, D), lambda i: (0, 0)),
        pl.BlockSpec((D,), lambda i: (0,)),
        pl.BlockSpec((D,), lambda i: (0,)),
        pl.BlockSpec((D,), lambda i: (0,)),
    ]
    args = (gathered.reshape(HB, S, D), condition, pos_emb, W_c, b_c,
            ln_scale, ln_bias)
    aliases = {}
    if o_prev is not None:
        specs = [pl.BlockSpec((8, 8, D), lambda i: (0, 0, 0))] + specs
        args = (o_prev,) + args
        aliases = {0: 0}
    return pl.pallas_call(
        _tc_body,
        grid=(HB // BB,),
        in_specs=specs,
        out_specs=pl.BlockSpec((BB, SO, D), lambda i: (hb0 + i, 0, 0)),
        out_shape=jax.ShapeDtypeStruct((B, SO, D), jnp.float32),
        input_output_aliases=aliases,
    )(*args)


def kernel(ids, condition, table, pos_emb, W_c, b_c, ln_scale, ln_bias):
    ids_flat = ids.reshape(B * S).astype(jnp.int32)
    g = [_sc_gather_half(ids_flat, table, c) for c in range(NCK)]
    o = _tc_epilogue_half(None, g[0], condition, pos_emb, W_c, b_c,
                          ln_scale, ln_bias, 0)
    for c in range(1, NCK):
        o = _tc_epilogue_half(o, g[c], condition, pos_emb, W_c, b_c,
                              ln_scale, ln_bias, c)
    return o


# speedup vs baseline: 1.0126x; 1.0126x over previous
import functools

import jax
import jax.numpy as jnp
from jax import lax
from jax.experimental import pallas as pl
from jax.experimental.pallas import tpu as pltpu
from jax.experimental.pallas import tpu_sc as plsc

B = 1024
S = 200
SO = S + 1
V = 100000
D = 128
CD = 128
MAXLEN = 256
EPS = 1e-12

NW = 32
NCK = 4
HALF_ROWS = B * S // NCK
ROWS_PER_W = HALF_ROWS // NW
CHUNK = 400
NCHUNK = ROWS_PER_W // CHUNK
BB = 64
HB = B // NCK


def _sc_gather_half(ids_flat, table, half):
    mesh = plsc.VectorSubcoreMesh(core_axis_name="c", subcore_axis_name="s")

    @functools.partial(
        pl.kernel,
        mesh=mesh,
        out_type=jax.ShapeDtypeStruct((HALF_ROWS, D), jnp.float32),
        scratch_types=[
            pltpu.VMEM((ROWS_PER_W,), jnp.int32),
            pltpu.VMEM((CHUNK, D), jnp.float32),
            pltpu.VMEM((CHUNK, D), jnp.float32),
            pltpu.SemaphoreType.DMA,
            pltpu.SemaphoreType.DMA,
            pltpu.SemaphoreType.DMA,
            pltpu.SemaphoreType.DMA,
        ],
    )
    def k(ids_hbm, table_hbm, out_hbm, idx_v, rows0, rows1, g0, g1, s0, s1):
        cid = lax.axis_index("c")
        sid = lax.axis_index("s")
        wid = sid * 2 + cid
        base = wid * ROWS_PER_W
        bufs = (rows0, rows1)
        gsems = (g0, g1)
        ssems = (s0, s1)
        pltpu.sync_copy(
            ids_hbm.at[pl.ds(half * HALF_ROWS + base, ROWS_PER_W)], idx_v)
        cpg = [None, None]
        cps = [None, None]
        cpg[0] = pltpu.async_copy(
            table_hbm.at[idx_v.at[pl.ds(0, CHUNK)]], bufs[0], gsems[0])
        for c in range(NCHUNK):
            p = c % 2
            if c + 1 < NCHUNK:
                q = 1 - p
                if cps[q] is not None:
                    cps[q].wait()
                cpg[q] = pltpu.async_copy(
                    table_hbm.at[idx_v.at[pl.ds((c + 1) * CHUNK, CHUNK)]],
                    bufs[q], gsems[q])
            cpg[p].wait()
            cps[p] = pltpu.async_copy(
                bufs[p], out_hbm.at[pl.ds(base + c * CHUNK, CHUNK)], ssems[p])
        cps[0].wait()
        cps[1].wait()

    return k(ids_flat, table)


def _tc_body(*refs):
    if len(refs) == 9:
        refs = refs[1:]
    g_ref, cond_ref, pos_ref, wc_ref, bc_ref, sc_ref, bi_ref, o_ref = refs
    g = g_ref[...]
    cond = cond_ref[:, 0, :]
    ce = jnp.dot(cond, wc_ref[...],
                 preferred_element_type=jnp.float32) + bc_ref[...]
    x = jnp.concatenate(
        [jnp.zeros((BB, 1, D), jnp.float32), g], axis=1)
    x = x + pos_ref[0, :SO, :][None, :, :] + ce[:, None, :]
    mean = jnp.mean(x, axis=-1, keepdims=True)
    var = jnp.mean(jnp.square(x), axis=-1, keepdims=True) - jnp.square(mean)
    y = (x - mean) * lax.rsqrt(var + EPS)
    o_ref[...] = y * sc_ref[...][None, None, :] + bi_ref[...][None, None, :]


def _tc_epilogue_half(o_prev, gathered, condition, pos_emb, W_c, b_c,
                      ln_scale, ln_bias, half):
    hb0 = half * (HB // BB)
    specs = [
        pl.BlockSpec((BB, S, D), lambda i: (i, 0, 0)),
        pl.BlockSpec((BB, 1, CD), lambda i: (hb0 + i, 0, 0)),
        pl.BlockSpec((1, MAXLEN, D), lambda i: (0, 0, 0)),
        pl.BlockSpec((CD, D), lambda i: (0, 0)),
        pl.BlockSpec((D,), lambda i: (0,)),
        pl.BlockSpec((D,), lambda i: (0,)),
        pl.BlockSpec((D,), lambda i: (0,)),
    ]
    args = (gathered.reshape(HB, S, D), condition, pos_emb, W_c, b_c,
            ln_scale, ln_bias)
    aliases = {}
    if o_prev is not None:
        specs = [pl.BlockSpec((8, 8, D), lambda i: (0, 0, 0))] + specs
        args = (o_prev,) + args
        aliases = {0: 0}
    return pl.pallas_call(
        _tc_body,
        grid=(HB // BB,),
        in_specs=specs,
        out_specs=pl.BlockSpec((BB, SO, D), lambda i: (hb0 + i, 0, 0)),
        out_shape=jax.ShapeDtypeStruct((B, SO, D), jnp.float32),
        input_output_aliases=aliases,
    )(*args)


def kernel(ids, condition, table, pos_emb, W_c, b_c, ln_scale, ln_bias):
    ids_flat = ids.reshape(B * S).astype(jnp.int32)
    g = [_sc_gather_half(ids_flat, table, c) for c in range(NCK)]
    o = _tc_epilogue_half(None, g[0], condition, pos_emb, W_c, b_c,
                          ln_scale, ln_bias, 0)
    for c in range(1, NCK):
        o = _tc_epilogue_half(o, g[c], condition, pos_emb, W_c, b_c,
                              ln_scale, ln_bias, c)
    return o
